# baseline (speedup 1.0000x reference)
import functools

import jax
import jax.numpy as jnp
from jax import lax
from jax.experimental import pallas as pl
from jax.experimental.pallas import tpu as pltpu
from jax.experimental.pallas import tpu_sc as plsc

BATCH = 16384
EMBED_DIM = 64
NUM_VOCAB = 1000000
NUM_WORKERS = 32
B_PER_W = BATCH // NUM_WORKERS
LANES = 16
N_GROUPS = B_PER_W // LANES
RING = 2
TAIL_START = (NUM_VOCAB // 128) * 128
TAIL_LEN = NUM_VOCAB - TAIL_START
LAST_FULL_TILE = NUM_VOCAB // 128 - 2


def _sc_body(uid_hbm, iid_hbm, utab_hbm, itab_hbm, out_hbm,
             uidx_v, iidx_v, ublk_v, iblk_v, uslab_v, islab_v, out_v,
             usem, isem):
    wid = lax.axis_index("s") * 2 + lax.axis_index("c")
    base = wid * B_PER_W

    pltpu.sync_copy(uid_hbm.at[pl.ds(base, B_PER_W)], uidx_v)
    pltpu.sync_copy(iid_hbm.at[pl.ds(base, B_PER_W)], iidx_v)
    pltpu.sync_copy(utab_hbm.at[:, pl.ds(TAIL_START, TAIL_LEN)], uslab_v)
    pltpu.sync_copy(itab_hbm.at[:, pl.ds(TAIL_START, TAIL_LEN)], islab_v)

    lane = lax.iota(jnp.int32, LANES)

    def issue(table_hbm, blk_v, sem, ident, slot):
        tl = lax.shift_right_logical(ident, 7)
        tl_c = jnp.minimum(tl, LAST_FULL_TILE)
        off = pl.multiple_of(tl_c * 128, 128)
        pltpu.async_copy(table_hbm.at[:, pl.ds(off, 256)],
                         blk_v.at[slot], sem.at[slot])

    def drain(table_hbm, blk_v, sem, slot):
        pltpu.make_async_copy(table_hbm.at[:, pl.ds(0, 256)],
                              blk_v.at[slot], sem.at[slot]).wait()

    def col(blk2d, slab2d, ident):
        tl_c = jnp.minimum(lax.shift_right_logical(ident, 7), LAST_FULL_TILE)
        lam = jnp.minimum(ident - tl_c * 128, 255)
        lam_s = jnp.clip(ident - TAIL_START, 0, TAIL_LEN - 1)
        is_tail = lax.broadcast(ident >= TAIL_START, (LANES,))
        b_lam = lax.broadcast(lam, (LANES,))
        b_lam_s = lax.broadcast(lam_s, (LANES,))
        parts = []
        for q in range(EMBED_DIM // LANES):
            rows = lane + q * LANES
            v_blk = plsc.load_gather(blk2d, [rows, b_lam])
            v_slab = plsc.load_gather(slab2d, [rows, b_lam_s])
            parts.append(jnp.where(is_tail, v_slab, v_blk))
        return parts

    def load_ids(g):
        return uidx_v[pl.ds(g * LANES, LANES)], iidx_v[pl.ds(g * LANES, LANES)]

    def run_group(j, uids, iids, uids_n, iids_n, has_next):
        dotv = jnp.zeros((LANES,), jnp.float32)
        for e in range(LANES):
            slot = (4 * j + e) % RING
            drain(utab_hbm, ublk_v, usem, slot)
            drain(itab_hbm, iblk_v, isem, slot)
            u_parts = col(ublk_v.at[slot], uslab_v, uids[e])
            i_parts = col(iblk_v.at[slot], islab_v, iids[e])
            acc = u_parts[0] * i_parts[0]
            for q in range(1, EMBED_DIM // LANES):
                acc = acc + u_parts[q] * i_parts[q]
            dot = jnp.sum(acc)
            dotv = jnp.where(lane == e, lax.broadcast(dot, (LANES,)), dotv)
            if e + RING < LANES:
                u_next, i_next = uids[e + RING], iids[e + RING]
                do_issue = True
            elif uids_n is not None:
                u_next = uids_n[e + RING - LANES]
                i_next = iids_n[e + RING - LANES]
                do_issue = has_next
            else:
                do_issue = False
            if do_issue is not False:
                @pl.when(jnp.asarray(do_issue))
                def _():
                    issue(utab_hbm, ublk_v, usem, u_next, slot)
                    issue(itab_hbm, iblk_v, isem, i_next, slot)
        return dotv

    uids0, iids0 = load_ids(0)
    for r in range(RING):
        issue(utab_hbm, ublk_v, usem, uids0[r], r)
        issue(itab_hbm, iblk_v, isem, iids0[r], r)

    def super_group(k2, carry):
        g0 = 3 * k2
        ids = [load_ids(g0), load_ids(g0 + 1), load_ids(g0 + 2),
               load_ids(jnp.minimum(g0 + 3, N_GROUPS - 1))]
        for j in range(3):
            dotv = run_group(j, *ids[j], *ids[j + 1],
                             has_next=(g0 + j + 1 <= N_GROUPS - 1)
                             if j < 2 else (g0 + 3 <= N_GROUPS - 1))
            out_v[pl.ds((g0 + j) * LANES, LANES)] = dotv
        return carry

    lax.fori_loop(0, (N_GROUPS - 2) // 3, super_group, 0)

    for g in (N_GROUPS - 2, N_GROUPS - 1):
        uids, iids = load_ids(g)
        if g + 1 <= N_GROUPS - 1:
            nxt = load_ids(g + 1)
        else:
            nxt = (None, None)
        dotv = run_group(g % 3, uids, iids, *nxt,
                         has_next=(g + 1 <= N_GROUPS - 1))
        out_v[pl.ds(g * LANES, LANES)] = dotv

    pltpu.sync_copy(out_v, out_hbm.at[pl.ds(base, B_PER_W)])


@jax.jit
def _mf_dot(user_ids, item_ids, user_table_t, item_table_t):
    mesh = plsc.VectorSubcoreMesh(core_axis_name="c", subcore_axis_name="s")
    return pl.kernel(
        _sc_body,
        mesh=mesh,
        compiler_params=pltpu.CompilerParams(needs_layout_passes=False),
        out_type=jax.ShapeDtypeStruct((BATCH,), jnp.float32),
        scratch_types=[
            pltpu.VMEM((B_PER_W,), jnp.int32),
            pltpu.VMEM((B_PER_W,), jnp.int32),
            pltpu.VMEM((RING, EMBED_DIM, 256), jnp.float32),
            pltpu.VMEM((RING, EMBED_DIM, 256), jnp.float32),
            pltpu.VMEM((EMBED_DIM, TAIL_LEN), jnp.float32),
            pltpu.VMEM((EMBED_DIM, TAIL_LEN), jnp.float32),
            pltpu.VMEM((B_PER_W,), jnp.float32),
            pltpu.SemaphoreType.DMA((RING,)),
            pltpu.SemaphoreType.DMA((RING,)),
        ],
    )(user_ids, item_ids, user_table_t, item_table_t)


def kernel(user_ids, item_ids, user_table, item_table):
    return _mf_dot(user_ids.astype(jnp.int32), item_ids.astype(jnp.int32),
                   user_table.T, item_table.T)


# speedup vs baseline: 2.1737x; 2.1737x over previous
import functools

import jax
import jax.numpy as jnp
from jax import lax
from jax.experimental import pallas as pl
from jax.experimental.pallas import tpu as pltpu
from jax.experimental.pallas import tpu_sc as plsc

BATCH = 16384
EMBED_DIM = 64
NUM_VOCAB = 1000000
NUM_WORKERS = 32
B_PER_W = BATCH // NUM_WORKERS
LANES = 16
N_GROUPS = B_PER_W // LANES
RING = 6
TAIL_START = (NUM_VOCAB // 128) * 128
TAIL_LEN = NUM_VOCAB - TAIL_START
LAST_FULL_TILE = NUM_VOCAB // 128 - 1


def _sc_body(uid_hbm, iid_hbm, utab_hbm, itab_hbm, out_hbm,
             uidx_v, iidx_v, ublk_v, iblk_v, uslab_v, islab_v, out_v,
             usem, isem):
    wid = lax.axis_index("s") * 2 + lax.axis_index("c")
    base = wid * B_PER_W

    pltpu.sync_copy(uid_hbm.at[pl.ds(base, B_PER_W)], uidx_v)
    pltpu.sync_copy(iid_hbm.at[pl.ds(base, B_PER_W)], iidx_v)
    pltpu.sync_copy(utab_hbm.at[:, pl.ds(TAIL_START, TAIL_LEN)], uslab_v)
    pltpu.sync_copy(itab_hbm.at[:, pl.ds(TAIL_START, TAIL_LEN)], islab_v)

    lane = lax.iota(jnp.int32, LANES)

    def issue(table_hbm, blk_v, sem, ident, slot):
        tl = lax.shift_right_logical(ident, 7)
        tl_c = jnp.minimum(tl, LAST_FULL_TILE)
        off = pl.multiple_of(tl_c * 128, 128)
        pltpu.async_copy(table_hbm.at[:, pl.ds(off, 128)],
                         blk_v.at[slot], sem.at[slot])

    def drain(table_hbm, blk_v, sem, slot):
        pltpu.make_async_copy(table_hbm.at[:, pl.ds(0, 128)],
                              blk_v.at[slot], sem.at[slot]).wait()

    def col(blk2d, slab2d, ident):
        tl_c = jnp.minimum(lax.shift_right_logical(ident, 7), LAST_FULL_TILE)
        lam = jnp.minimum(ident - tl_c * 128, 127)
        lam_s = jnp.clip(ident - TAIL_START, 0, TAIL_LEN - 1)
        is_tail = lax.broadcast(ident >= TAIL_START, (LANES,))
        b_lam = lax.broadcast(lam, (LANES,))
        b_lam_s = lax.broadcast(lam_s, (LANES,))
        parts = []
        for q in range(EMBED_DIM // LANES):
            rows = lane + q * LANES
            v_blk = plsc.load_gather(blk2d, [rows, b_lam])
            v_slab = plsc.load_gather(slab2d, [rows, b_lam_s])
            parts.append(jnp.where(is_tail, v_slab, v_blk))
        return parts

    def load_ids(g):
        return uidx_v[pl.ds(g * LANES, LANES)], iidx_v[pl.ds(g * LANES, LANES)]

    def run_group(j, uids, iids, uids_n, iids_n, has_next):
        dotv = jnp.zeros((LANES,), jnp.float32)
        for e in range(LANES):
            slot = (4 * j + e) % RING
            drain(utab_hbm, ublk_v, usem, slot)
            drain(itab_hbm, iblk_v, isem, slot)
            u_parts = col(ublk_v.at[slot], uslab_v, uids[e])
            i_parts = col(iblk_v.at[slot], islab_v, iids[e])
            acc = u_parts[0] * i_parts[0]
            for q in range(1, EMBED_DIM // LANES):
                acc = acc + u_parts[q] * i_parts[q]
            dot = jnp.sum(acc)
            dotv = jnp.where(lane == e, lax.broadcast(dot, (LANES,)), dotv)
            if e + RING < LANES:
                u_next, i_next = uids[e + RING], iids[e + RING]
                do_issue = True
            elif uids_n is not None:
                u_next = uids_n[e + RING - LANES]
                i_next = iids_n[e + RING - LANES]
                do_issue = has_next
            else:
                do_issue = False
            if do_issue is not False:
                @pl.when(jnp.asarray(do_issue))
                def _():
                    issue(utab_hbm, ublk_v, usem, u_next, slot)
                    issue(itab_hbm, iblk_v, isem, i_next, slot)
        return dotv

    uids0, iids0 = load_ids(0)
    for r in range(RING):
        issue(utab_hbm, ublk_v, usem, uids0[r], r)
        issue(itab_hbm, iblk_v, isem, iids0[r], r)

    def super_group(k2, carry):
        g0 = 3 * k2
        ids = [load_ids(g0), load_ids(g0 + 1), load_ids(g0 + 2),
               load_ids(jnp.minimum(g0 + 3, N_GROUPS - 1))]
        for j in range(3):
            dotv = run_group(j, *ids[j], *ids[j + 1],
                             has_next=(g0 + j + 1 <= N_GROUPS - 1)
                             if j < 2 else (g0 + 3 <= N_GROUPS - 1))
            out_v[pl.ds((g0 + j) * LANES, LANES)] = dotv
        return carry

    lax.fori_loop(0, (N_GROUPS - 2) // 3, super_group, 0)

    for g in (N_GROUPS - 2, N_GROUPS - 1):
        uids, iids = load_ids(g)
        if g + 1 <= N_GROUPS - 1:
            nxt = load_ids(g + 1)
        else:
            nxt = (None, None)
        dotv = run_group(g % 3, uids, iids, *nxt,
                         has_next=(g + 1 <= N_GROUPS - 1))
        out_v[pl.ds(g * LANES, LANES)] = dotv

    pltpu.sync_copy(out_v, out_hbm.at[pl.ds(base, B_PER_W)])


@jax.jit
def _mf_dot(user_ids, item_ids, user_table_t, item_table_t):
    mesh = plsc.VectorSubcoreMesh(core_axis_name="c", subcore_axis_name="s")
    return pl.kernel(
        _sc_body,
        mesh=mesh,
        compiler_params=pltpu.CompilerParams(needs_layout_passes=False),
        out_type=jax.ShapeDtypeStruct((BATCH,), jnp.float32),
        scratch_types=[
            pltpu.VMEM((B_PER_W,), jnp.int32),
            pltpu.VMEM((B_PER_W,), jnp.int32),
            pltpu.VMEM((RING, EMBED_DIM, 128), jnp.float32),
            pltpu.VMEM((RING, EMBED_DIM, 128), jnp.float32),
            pltpu.VMEM((EMBED_DIM, TAIL_LEN), jnp.float32),
            pltpu.VMEM((EMBED_DIM, TAIL_LEN), jnp.float32),
            pltpu.VMEM((B_PER_W,), jnp.float32),
            pltpu.SemaphoreType.DMA((RING,)),
            pltpu.SemaphoreType.DMA((RING,)),
        ],
    )(user_ids, item_ids, user_table_t, item_table_t)


def kernel(user_ids, item_ids, user_table, item_table):
    return _mf_dot(user_ids.astype(jnp.int32), item_ids.astype(jnp.int32),
                   user_table.T, item_table.T)
